# baseline (speedup 1.0000x reference)
import functools
import math

import jax
import jax.numpy as jnp
from jax import lax
from jax.experimental import pallas as pl
from jax.experimental.pallas import tpu as pltpu
from jax.experimental.pallas import tpu_sc as plsc

_NC = 2
_NS = 16
_C = 128
_G = 16
_BLK = 80
_F0 = 0.5


def _sc_aggregate(hb, src0, dst0, src1, dst1, n_pad, ch0, ch1, d):
    dw = d // 2
    rows_per_sub = n_pad // _NS
    full = rows_per_sub // _C
    assert rows_per_sub % _C == 0

    mesh = plsc.VectorSubcoreMesh(core_axis_name="c", subcore_axis_name="s")

    @functools.partial(
        pl.kernel,
        out_type=jax.ShapeDtypeStruct((_NC * n_pad, d), jnp.float32),
        mesh=mesh,
        compiler_params=pltpu.CompilerParams(use_tc_tiling_on_sc=False),
        scratch_types=[
            pltpu.VMEM((2 * _G, _C), jnp.int32),
            pltpu.VMEM((2 * _G, _C), jnp.int32),
            pltpu.VMEM((_C, dw), jnp.int32),
            pltpu.VMEM((_C, dw), jnp.int32),
            pltpu.VMEM((_C, d), jnp.float32),
            pltpu.VMEM_SHARED((n_pad, d), jnp.float32),
            pltpu.SemaphoreType.DMA,
            pltpu.SemaphoreType.DMA,
        ],
    )
    def agg(hb_hbm, src0_hbm, dst0_hbm, src1_hbm, dst1_hbm, out_hbm,
            sidx, didx, gbuf0, gbuf1, fbuf, acc_sh, semg0, semg1):
        c = lax.axis_index("c")
        s = lax.axis_index("s")

        zval = jnp.zeros((16,), jnp.float32)

        def zero_body(i, carry):
            for k in range(d // 16):
                fbuf[i, pl.ds(k * 16, 16)] = zval
            return carry

        lax.fori_loop(0, _C, zero_body, 0)
        base = s * rows_per_sub
        for t in range(full):
            pltpu.sync_copy(fbuf, acc_sh.at[pl.ds(base + t * _C, _C)])
        plsc.subcore_barrier()

        gbufs = (gbuf0, gbuf1)
        sems = (semg0, semg1)

        def expand_chunk(gbuf):
            def conv_body(r2, carry):
                for u in range(2):
                    r = 2 * r2 + u
                    for k in range(dw // 16):
                        wv = gbuf[r, pl.ds(k * 16, 16)]
                        lo = lax.bitcast_convert_type(wv << 16, jnp.float32)
                        hi = lax.bitcast_convert_type(wv & jnp.int32(-65536),
                                                      jnp.float32)
                        fbuf[r, pl.ds(k * 32, 16)] = lo
                        fbuf[r, pl.ds(k * 32 + 16, 16)] = hi
                return carry

            lax.fori_loop(0, _C // 2, conv_body, 0)

        def run_core(src_hbm, dst_hbm, ch_c):
            ngroups = ch_c // _G
            pltpu.sync_copy(src_hbm.at[s, pl.ds(0, _G)], sidx.at[pl.ds(0, _G)])
            pltpu.sync_copy(dst_hbm.at[s, pl.ds(0, _G)], didx.at[pl.ds(0, _G)])
            pltpu.async_copy(hb_hbm.at[sidx.at[0]], gbuf0, semg0)

            def chunk_body(j, carry):
                g = j // _G
                r = (g % 2) * _G + (j - g * _G)

                @pl.when(jnp.logical_and(j == g * _G, g + 1 < ngroups))
                def _prefetch_idx():
                    half = ((g + 1) % 2) * _G
                    pltpu.sync_copy(src_hbm.at[s, pl.ds((g + 1) * _G, _G)],
                                    sidx.at[pl.ds(half, _G)])
                    pltpu.sync_copy(dst_hbm.at[s, pl.ds((g + 1) * _G, _G)],
                                    didx.at[pl.ds(half, _G)])

                for p in range(2):
                    @pl.when(j % 2 == p)
                    def _step():
                        pltpu.make_async_copy(hb_hbm.at[sidx.at[r]],
                                              gbufs[p], sems[p]).wait()

                        @pl.when(j + 1 < ch_c)
                        def _next_gather():
                            j2 = j + 1
                            g2 = j2 // _G
                            r2 = (g2 % 2) * _G + (j2 - g2 * _G)
                            pltpu.async_copy(hb_hbm.at[sidx.at[r2]],
                                             gbufs[1 - p], sems[1 - p])

                        expand_chunk(gbufs[p])
                        pltpu.sync_copy(fbuf, acc_sh.at[didx.at[r]], add=True)

                return carry

            lax.fori_loop(0, ch_c, chunk_body, 0)

        @pl.when(c == 0)
        def _core0():
            run_core(src0_hbm, dst0_hbm, ch0)

        @pl.when(c == 1)
        def _core1():
            run_core(src1_hbm, dst1_hbm, ch1)

        plsc.subcore_barrier()

        pltpu.sync_copy(acc_sh.at[pl.ds(base, rows_per_sub)],
                        out_hbm.at[pl.ds(c * n_pad + base, rows_per_sub)])

    return agg(hb, src0, dst0, src1, dst1)


def _tc_transform(parts, w_mat, b_row, n, n_pad, packed):
    d = w_mat.shape[0]
    nb = n // _BLK
    off = n_pad // _BLK

    def body(p0_ref, p1_ref, w_ref, b_ref, o_ref):
        a = p0_ref[...] + p1_ref[...]
        y = jnp.dot(a, w_ref[...], preferred_element_type=jnp.float32) + b_ref[...]
        y = y * (1.0 / (1.0 + jnp.exp(-y)))
        if not packed:
            o_ref[...] = y
            return
        yb = jax.lax.bitcast_convert_type(
            y.astype(jnp.bfloat16), jnp.uint16).astype(jnp.uint32)
        for k in range(d // 32):
            lo = yb[:, k * 32:k * 32 + 16]
            hi = yb[:, k * 32 + 16:k * 32 + 32]
            o_ref[:, k * 16:(k + 1) * 16] = jax.lax.bitcast_convert_type(
                lo | (hi << 16), jnp.int32)

    out_d = d // 2 if packed else d
    out_t = jnp.int32 if packed else jnp.float32
    return pl.pallas_call(
        body,
        grid=(nb,),
        in_specs=[
            pl.BlockSpec((_BLK, d), lambda i: (i, 0)),
            pl.BlockSpec((_BLK, d), lambda i: (i + off, 0)),
            pl.BlockSpec((d, d), lambda i: (0, 0)),
            pl.BlockSpec((1, d), lambda i: (0, 0)),
        ],
        out_specs=pl.BlockSpec((_BLK, out_d), lambda i: (i, 0)),
        out_shape=jax.ShapeDtypeStruct((n, out_d), out_t),
    )(parts, parts, w_mat, b_row)


def _pack_swizzled(h, n, d):
    hb = h.astype(jnp.bfloat16).reshape(n, d // 32, 2, 16).swapaxes(2, 3)
    return jax.lax.bitcast_convert_type(hb, jnp.int32).reshape(n, d // 2)


def kernel(x, edge_index, W, b):
    n, d = x.shape
    e = edge_index.shape[1]
    assert n % _BLK == 0 and d % 32 == 0
    lcm = _BLK * 128 // math.gcd(_BLK, 128)
    n_pad = ((n // lcm) + 1) * lcm

    tot = -(-e // (_NS * _C))
    ch0 = max(0, int(round(_F0 * tot / _G))) * _G
    ch1 = max(0, -(-(tot - ch0) // _G)) * _G
    cap = _NS * _C * (ch0 + ch1)
    pad = cap - e
    src = edge_index[0]
    dst = edge_index[1]
    if pad:
        src = jnp.concatenate([src, jnp.zeros((pad,), jnp.int32)])
        dst = jnp.concatenate([dst, jnp.full((pad,), n, jnp.int32)])
    n0 = _NS * ch0 * _C

    def blockify(v, ch_c, lo, hi):
        if ch_c == 0:
            return jnp.zeros((_NS, _G, _C), jnp.int32)
        return v[lo:hi].reshape(_NS, ch_c, _C)

    src0 = blockify(src, ch0, 0, n0)
    dst0 = blockify(dst, ch0, 0, n0)
    src1 = blockify(src, ch1, n0, cap)
    dst1 = blockify(dst, ch1, n0, cap)
    b_row = b.reshape(1, d)

    hb = _pack_swizzled(x, n, d)
    parts = _sc_aggregate(hb, src0, dst0, src1, dst1, n_pad, ch0, ch1, d)
    hb = _tc_transform(parts, W, b_row, n, n_pad, packed=True)
    parts = _sc_aggregate(hb, src0, dst0, src1, dst1, n_pad, ch0, ch1, d)
    return _tc_transform(parts, W, b_row, n, n_pad, packed=False)


# speedup vs baseline: 1.0032x; 1.0032x over previous
import functools
import math

import jax
import jax.numpy as jnp
from jax import lax
from jax.experimental import pallas as pl
from jax.experimental.pallas import tpu as pltpu
from jax.experimental.pallas import tpu_sc as plsc

_NC = 2
_NS = 16
_C = 128
_G = 16
_BLK = 80
_F0 = 0.5


def _sc_aggregate(hb, src0, dst0, src1, dst1, n_pad, ch0, ch1, d):
    dw = d // 2
    rows_per_sub = n_pad // _NS
    full = rows_per_sub // _C
    assert rows_per_sub % _C == 0

    mesh = plsc.VectorSubcoreMesh(core_axis_name="c", subcore_axis_name="s")

    @functools.partial(
        pl.kernel,
        out_type=jax.ShapeDtypeStruct((_NC * n_pad, d), jnp.float32),
        mesh=mesh,
        compiler_params=pltpu.CompilerParams(use_tc_tiling_on_sc=False),
        scratch_types=[
            pltpu.VMEM((2 * _G, _C), jnp.int32),
            pltpu.VMEM((2 * _G, _C), jnp.int32),
            pltpu.VMEM((_C, dw), jnp.int32),
            pltpu.VMEM((_C, dw), jnp.int32),
            pltpu.VMEM((_C, d), jnp.float32),
            pltpu.VMEM_SHARED((n_pad, d), jnp.float32),
            pltpu.SemaphoreType.DMA,
            pltpu.SemaphoreType.DMA,
        ],
    )
    def agg(hb_hbm, src0_hbm, dst0_hbm, src1_hbm, dst1_hbm, out_hbm,
            sidx, didx, gbuf0, gbuf1, fbuf, acc_sh, semg0, semg1):
        c = lax.axis_index("c")
        s = lax.axis_index("s")

        zval = jnp.zeros((16,), jnp.float32)

        def zero_body(i, carry):
            for k in range(d // 16):
                fbuf[i, pl.ds(k * 16, 16)] = zval
            return carry

        lax.fori_loop(0, _C, zero_body, 0)
        base = s * rows_per_sub
        for t in range(full):
            pltpu.sync_copy(fbuf, acc_sh.at[pl.ds(base + t * _C, _C)])
        plsc.subcore_barrier()

        gbufs = (gbuf0, gbuf1)
        sems = (semg0, semg1)

        def expand_chunk(gbuf):
            def conv_body(r, carry):
                for k in range(dw // 16):
                    wv = gbuf[r, pl.ds(k * 16, 16)]
                    lo = lax.bitcast_convert_type(wv << 16, jnp.float32)
                    hi = lax.bitcast_convert_type(wv & jnp.int32(-65536),
                                                  jnp.float32)
                    fbuf[r, pl.ds(k * 32, 16)] = lo
                    fbuf[r, pl.ds(k * 32 + 16, 16)] = hi
                return carry

            lax.fori_loop(0, _C, conv_body, 0)

        def run_core(src_hbm, dst_hbm, ch_c):
            ngroups = ch_c // _G
            pltpu.sync_copy(src_hbm.at[s, pl.ds(0, _G)], sidx.at[pl.ds(0, _G)])
            pltpu.sync_copy(dst_hbm.at[s, pl.ds(0, _G)], didx.at[pl.ds(0, _G)])
            pltpu.async_copy(hb_hbm.at[sidx.at[0]], gbuf0, semg0)

            def chunk_body(j, carry):
                g = j // _G
                r = (g % 2) * _G + (j - g * _G)

                @pl.when(jnp.logical_and(j == g * _G, g + 1 < ngroups))
                def _prefetch_idx():
                    half = ((g + 1) % 2) * _G
                    pltpu.sync_copy(src_hbm.at[s, pl.ds((g + 1) * _G, _G)],
                                    sidx.at[pl.ds(half, _G)])
                    pltpu.sync_copy(dst_hbm.at[s, pl.ds((g + 1) * _G, _G)],
                                    didx.at[pl.ds(half, _G)])

                for p in range(2):
                    @pl.when(j % 2 == p)
                    def _step():
                        pltpu.make_async_copy(hb_hbm.at[sidx.at[r]],
                                              gbufs[p], sems[p]).wait()

                        @pl.when(j + 1 < ch_c)
                        def _next_gather():
                            j2 = j + 1
                            g2 = j2 // _G
                            r2 = (g2 % 2) * _G + (j2 - g2 * _G)
                            pltpu.async_copy(hb_hbm.at[sidx.at[r2]],
                                             gbufs[1 - p], sems[1 - p])

                        expand_chunk(gbufs[p])
                        pltpu.sync_copy(fbuf, acc_sh.at[didx.at[r]], add=True)

                return carry

            lax.fori_loop(0, ch_c, chunk_body, 0)

        @pl.when(c == 0)
        def _core0():
            run_core(src0_hbm, dst0_hbm, ch0)

        @pl.when(c == 1)
        def _core1():
            run_core(src1_hbm, dst1_hbm, ch1)

        plsc.subcore_barrier()

        pltpu.sync_copy(acc_sh.at[pl.ds(base, rows_per_sub)],
                        out_hbm.at[pl.ds(c * n_pad + base, rows_per_sub)])

    return agg(hb, src0, dst0, src1, dst1)


def _tc_transform(parts, w_mat, b_row, n, n_pad, packed):
    d = w_mat.shape[0]
    nb = n // _BLK
    off = n_pad // _BLK

    def body(p0_ref, p1_ref, w_ref, b_ref, o_ref):
        a = p0_ref[...] + p1_ref[...]
        y = jnp.dot(a, w_ref[...], preferred_element_type=jnp.float32) + b_ref[...]
        y = y * (1.0 / (1.0 + jnp.exp(-y)))
        if not packed:
            o_ref[...] = y
            return
        yb = jax.lax.bitcast_convert_type(
            y.astype(jnp.bfloat16), jnp.uint16).astype(jnp.uint32)
        for k in range(d // 32):
            lo = yb[:, k * 32:k * 32 + 16]
            hi = yb[:, k * 32 + 16:k * 32 + 32]
            o_ref[:, k * 16:(k + 1) * 16] = jax.lax.bitcast_convert_type(
                lo | (hi << 16), jnp.int32)

    out_d = d // 2 if packed else d
    out_t = jnp.int32 if packed else jnp.float32
    return pl.pallas_call(
        body,
        grid=(nb,),
        in_specs=[
            pl.BlockSpec((_BLK, d), lambda i: (i, 0)),
            pl.BlockSpec((_BLK, d), lambda i: (i + off, 0)),
            pl.BlockSpec((d, d), lambda i: (0, 0)),
            pl.BlockSpec((1, d), lambda i: (0, 0)),
        ],
        out_specs=pl.BlockSpec((_BLK, out_d), lambda i: (i, 0)),
        out_shape=jax.ShapeDtypeStruct((n, out_d), out_t),
    )(parts, parts, w_mat, b_row)


def _pack_swizzled(h, n, d):
    hb = h.astype(jnp.bfloat16).reshape(n, d // 32, 2, 16).swapaxes(2, 3)
    return jax.lax.bitcast_convert_type(hb, jnp.int32).reshape(n, d // 2)


def kernel(x, edge_index, W, b):
    n, d = x.shape
    e = edge_index.shape[1]
    assert n % _BLK == 0 and d % 32 == 0
    lcm = _BLK * 128 // math.gcd(_BLK, 128)
    n_pad = ((n // lcm) + 1) * lcm

    tot = -(-e // (_NS * _C))
    ch0 = max(0, int(round(_F0 * tot / _G))) * _G
    ch1 = max(0, -(-(tot - ch0) // _G)) * _G
    cap = _NS * _C * (ch0 + ch1)
    pad = cap - e
    src = edge_index[0]
    dst = edge_index[1]
    if pad:
        src = jnp.concatenate([src, jnp.zeros((pad,), jnp.int32)])
        dst = jnp.concatenate([dst, jnp.full((pad,), n, jnp.int32)])
    n0 = _NS * ch0 * _C

    def blockify(v, ch_c, lo, hi):
        if ch_c == 0:
            return jnp.zeros((_NS, _G, _C), jnp.int32)
        return v[lo:hi].reshape(_NS, ch_c, _C)

    src0 = blockify(src, ch0, 0, n0)
    dst0 = blockify(dst, ch0, 0, n0)
    src1 = blockify(src, ch1, n0, cap)
    dst1 = blockify(dst, ch1, n0, cap)
    b_row = b.reshape(1, d)

    hb = _pack_swizzled(x, n, d)
    parts = _sc_aggregate(hb, src0, dst0, src1, dst1, n_pad, ch0, ch1, d)
    hb = _tc_transform(parts, W, b_row, n, n_pad, packed=True)
    parts = _sc_aggregate(hb, src0, dst0, src1, dst1, n_pad, ch0, ch1, d)
    return _tc_transform(parts, W, b_row, n, n_pad, packed=False)
